# baseline (speedup 1.0000x reference)
import functools

import jax
import jax.numpy as jnp
from jax import lax
from jax.experimental import pallas as pl
from jax.experimental.pallas import tpu as pltpu
from jax.experimental.pallas import tpu_sc as plsc

E = 320000
N_NODES = 50000
D_IN = 128
D_H = 128
D_O = 64
BLK = 2000

NCHUNK = 4
CHUNK_N = 12500
BAND = 12544
ACC_ROWS = 12560
DUMP = 12544
EPT = E // 16
SEG = 1000
NSEG = EPT // SEG
SEG_PAD = 1024
CLEN = 1024
ZROWS = BAND // 16
WB = 16


def _messages(src0, dst0, src1, dst1, w, code_table, provider_table):
    mesh = plsc.VectorSubcoreMesh(core_axis_name="c", subcore_axis_name="s")

    @functools.partial(
        pl.kernel,
        out_type=[jax.ShapeDtypeStruct((NCHUNK * BAND, 128), jnp.float32),
                  jax.ShapeDtypeStruct((NCHUNK * BAND, 128), jnp.float32)],
        mesh=mesh,
        compiler_params=pltpu.CompilerParams(needs_layout_passes=False,
                                             has_side_effects=True,
                                             use_tc_tiling_on_sc=False),
        scratch_types=[
            pltpu.VMEM((SEG_PAD,), jnp.int32),
            pltpu.VMEM((SEG_PAD,), jnp.int32),
            pltpu.VMEM((SEG_PAD,), jnp.float32),
            pltpu.VMEM((CLEN,), jnp.int32),
            pltpu.VMEM((CLEN,), jnp.int32),
            pltpu.VMEM((CLEN,), jnp.float32),
            pltpu.VMEM((64, 128), jnp.float32),
            pltpu.VMEM((64,), jnp.int32),
            pltpu.VMEM((16,), jnp.int32),
            pltpu.VMEM((16, 16), jnp.float32),
            pltpu.VMEM((1024,), jnp.int32),
            pltpu.VMEM((WB, 128), jnp.float32),
            pltpu.VMEM((WB, 16), jnp.float32),
            pltpu.VMEM_SHARED((ACC_ROWS, 128), jnp.float32),
            pltpu.VMEM_SHARED((ACC_ROWS, 16), jnp.float32),
            pltpu.SemaphoreType.DMA,
        ],
    )
    def k(src0_h, dst0_h, src1_h, dst1_h, w_h, ctab_h, ptab_h,
          out0_h, out1_h,
          src_s, dst_s, w_s, srcc, dstc, wc, gbuf, gidx, sidx, deg_stage,
          probe, nbuf, dbuf, acc_msg, acc_deg, sem):
        core = lax.axis_index("c")
        sub = lax.axis_index("s")
        iota = lax.iota(jnp.int32, 16)
        zeros16i = jnp.zeros((16,), jnp.int32)
        zeros16f = jnp.zeros((16,), jnp.float32)

        def init_srcc(i, c):
            srcc[pl.ds(i * 16, 16)] = zeros16i
            return c
        lax.fori_loop(0, CLEN // 16, init_srcc, 0)
        for t16 in (SEG, SEG_PAD - 16):
            dst_s[pl.ds(t16, 16)] = jnp.full((16,), -1, jnp.int32)
        for r in range(16):
            deg_stage[r, pl.ds(0, 16)] = zeros16f

        ebase = sub * EPT

        for d in range(2):
            sh = (src0_h, src1_h)[d]
            dh = (dst0_h, dst1_h)[d]
            th = (ctab_h, ptab_h)[d]
            oh = (out0_h, out1_h)[d]

            def task(local, carry):
                chunk = core * 2 + local
                chunk_lo = chunk * CHUNK_N
                band_lo = chunk * BAND
                if True:
                    for r2 in range(WB):
                        for q2 in range(8):
                            nbuf[r2, pl.ds(q2 * 16, 16)] = zeros16f
                        dbuf[r2, pl.ds(0, 16)] = zeros16f

                    def zb(b, cz):
                        r0 = sub * ZROWS + b * WB
                        pltpu.sync_copy(nbuf, acc_msg.at[pl.ds(r0, WB)])
                        pltpu.sync_copy(dbuf, acc_deg.at[pl.ds(r0, WB)])
                        return cz

                    lax.fori_loop(0, ZROWS // WB, zb, 0)
                plsc.subcore_barrier()

                def seg(s, c0):
                    sbase = ebase + s * SEG
                    pltpu.sync_copy(sh.at[pl.ds(sbase, SEG)],
                                    src_s.at[pl.ds(0, SEG)])
                    pltpu.sync_copy(dh.at[pl.ds(sbase, SEG)],
                                    dst_s.at[pl.ds(0, SEG)])
                    pltpu.sync_copy(w_h.at[pl.ds(sbase, SEG)],
                                    w_s.at[pl.ds(0, SEG)])

                    def compact(i, cnt):
                        off = i * 16
                        dv = dst_s[pl.ds(off, 16)] - chunk_lo
                        sv = src_s[pl.ds(off, 16)]
                        wv = w_s[pl.ds(off, 16)]
                        m = (dv >= 0) & (dv < CHUNK_N)
                        mi = jnp.where(m, 1, 0)
                        pos = cnt + plsc.cumsum(mi) - 1
                        plsc.store_scatter(dstc, [pos], dv, mask=m)
                        plsc.store_scatter(srcc, [pos], sv, mask=m)
                        plsc.store_scatter(wc, [pos], wv, mask=m)
                        return cnt + jnp.sum(mi)

                    cnt = lax.fori_loop(0, (SEG + 15) // 16, compact,
                                        jnp.int32(0))

                    def proc(j, c1):
                        jo64 = j * 64
                        for g in range(4):
                            gidx[pl.ds(g * 16, 16)] = srcc[
                                pl.ds(jo64 + g * 16, 16)]
                        pltpu.async_copy(th.at[gidx], gbuf, sem).wait()
                        for g in range(4):
                            jo = jo64 + g * 16
                            valid = (jo + iota) < cnt
                            idxv = jnp.where(valid, dstc[pl.ds(jo, 16)],
                                             DUMP)
                            sidx[pl.ds(0, 16)] = idxv
                            wvec = jnp.where(valid, wc[pl.ds(jo, 16)],
                                             jnp.float32(0.0))
                            for r in range(16):
                                w_r = wvec[r]
                                for q in range(8):
                                    gbuf[g * 16 + r, pl.ds(q * 16, 16)] = (
                                        gbuf[g * 16 + r, pl.ds(q * 16, 16)]
                                        * w_r)
                            plsc.store_scatter(deg_stage, [iota, zeros16i],
                                               wvec)
                            hidx = idxv & 1023
                            plsc.store_scatter(probe, [hidx], iota)
                            rb = plsc.load_gather(probe, [hidx])
                            dup = jnp.sum(jnp.where(rb != iota, 1, 0)) > 0

                            @pl.when(jnp.logical_not(dup))
                            def _():
                                pltpu.sync_copy(gbuf.at[pl.ds(g * 16, 16)],
                                                acc_msg.at[sidx], add=True)
                                pltpu.sync_copy(deg_stage, acc_deg.at[sidx],
                                                add=True)

                            @pl.when(dup)
                            def _():
                                def slow(r, cs):
                                    sidx[pl.ds(0, 16)] = jnp.where(
                                        iota == r, idxv, DUMP)
                                    plsc.store_scatter(
                                        deg_stage, [iota, zeros16i],
                                        jnp.where(iota == r, wvec,
                                                  jnp.float32(0.0)))
                                    pltpu.sync_copy(
                                        gbuf.at[pl.ds(g * 16, 16)],
                                        acc_msg.at[sidx], add=True)
                                    pltpu.sync_copy(deg_stage,
                                                    acc_deg.at[sidx],
                                                    add=True)
                                    return cs
                                lax.fori_loop(0, 16, slow, 0)
                        return c1

                    nblk = (cnt + 63) // 64
                    lax.fori_loop(0, nblk, proc, 0)
                    return c0

                lax.fori_loop(0, NSEG, seg, 0)
                plsc.subcore_barrier()

                def wb(b, c2):
                    r0 = sub * ZROWS + b * WB
                    pltpu.sync_copy(acc_msg.at[pl.ds(r0, WB)], nbuf)
                    pltpu.sync_copy(acc_deg.at[pl.ds(r0, WB)], dbuf)

                    def row(r, c3):
                        dv = dbuf[r, pl.ds(0, 16)]
                        invv = 1.0 / (dv + 1e-8)
                        inv = invv[0]
                        for q in range(8):
                            nbuf[r, pl.ds(q * 16, 16)] = (
                                nbuf[r, pl.ds(q * 16, 16)] * inv)
                        return c3

                    lax.fori_loop(0, WB, row, 0)
                    pltpu.sync_copy(nbuf, oh.at[pl.ds(band_lo + r0, WB)])
                    return c2

                lax.fori_loop(0, ZROWS // WB, wb, 0)
                plsc.subcore_barrier()
                return carry

            lax.fori_loop(0, 2, task, 0)

    return k(src0, dst0, src1, dst1, w, code_table, provider_table)


def _mlp_body(x_ref, m_ref, w1a_ref, w1b_ref, b1_ref, g1_ref, be1_ref,
              w2_ref, b2_ref, g2_ref, be2_ref, o_ref):
    x = x_ref[...]
    m = m_ref[...]
    h = (jnp.dot(x, w1a_ref[...], preferred_element_type=jnp.float32)
         + jnp.dot(m, w1b_ref[...], preferred_element_type=jnp.float32)
         + b1_ref[...])
    mu = jnp.mean(h, axis=-1, keepdims=True)
    var = jnp.mean((h - mu) ** 2, axis=-1, keepdims=True)
    h = (h - mu) / jnp.sqrt(var + 1e-5) * g1_ref[...] + be1_ref[...]
    h = jnp.maximum(h, 0.0)
    o = jnp.dot(h, w2_ref[...], preferred_element_type=jnp.float32) + b2_ref[...]
    mu2 = jnp.mean(o, axis=-1, keepdims=True)
    var2 = jnp.mean((o - mu2) ** 2, axis=-1, keepdims=True)
    o_ref[...] = (o - mu2) / jnp.sqrt(var2 + 1e-5) * g2_ref[...] + be2_ref[...]


def _mlp(x, m, W1, b1, g1, be1, W2, b2, g2, be2):
    w1a = W1[:D_IN]
    w1b = W1[D_IN:]
    row = lambda i: (i, 0)
    fixed = lambda i: (0, 0)
    return pl.pallas_call(
        _mlp_body,
        grid=(N_NODES // BLK,),
        in_specs=[
            pl.BlockSpec((BLK, D_IN), row),
            pl.BlockSpec((BLK, D_IN), row),
            pl.BlockSpec((D_IN, D_H), fixed),
            pl.BlockSpec((D_IN, D_H), fixed),
            pl.BlockSpec((1, D_H), fixed),
            pl.BlockSpec((1, D_H), fixed),
            pl.BlockSpec((1, D_H), fixed),
            pl.BlockSpec((D_H, D_O), fixed),
            pl.BlockSpec((1, D_O), fixed),
            pl.BlockSpec((1, D_O), fixed),
            pl.BlockSpec((1, D_O), fixed),
        ],
        out_specs=pl.BlockSpec((BLK, D_O), row),
        out_shape=jax.ShapeDtypeStruct((N_NODES, D_O), jnp.float32),
    )(x, m, w1a, w1b, b1.reshape(1, -1), g1.reshape(1, -1), be1.reshape(1, -1),
      W2, b2.reshape(1, -1), g2.reshape(1, -1), be2.reshape(1, -1))


def kernel(provider_code_edges, code_provider_edges, edge_weights,
           provider_table, code_table,
           pW1, pb1, pg1, pbe1, pW2, pb2, pg2, pbe2,
           cW1, cb1, cg1, cbe1, cW2, cb2, cg2, cbe2):
    src0 = code_provider_edges[0].astype(jnp.int32)
    dst0 = code_provider_edges[1].astype(jnp.int32)
    src1 = provider_code_edges[0].astype(jnp.int32)
    dst1 = provider_code_edges[1].astype(jnp.int32)
    prov_band, code_band = _messages(src0, dst0, src1, dst1, edge_weights,
                                     code_table, provider_table)
    prov_msg = prov_band.reshape(NCHUNK, BAND, D_IN)[:, :CHUNK_N].reshape(
        N_NODES, D_IN)
    code_msg = code_band.reshape(NCHUNK, BAND, D_IN)[:, :CHUNK_N].reshape(
        N_NODES, D_IN)
    provider_out = _mlp(provider_table, prov_msg,
                        pW1, pb1, pg1, pbe1, pW2, pb2, pg2, pbe2)
    code_out = _mlp(code_table, code_msg,
                    cW1, cb1, cg1, cbe1, cW2, cb2, cg2, cbe2)
    return (provider_out, code_out)


# speedup vs baseline: 1.1226x; 1.1226x over previous
import functools

import jax
import jax.numpy as jnp
from jax import lax
from jax.experimental import pallas as pl
from jax.experimental.pallas import tpu as pltpu
from jax.experimental.pallas import tpu_sc as plsc

E = 320000
N_NODES = 50000
D_IN = 128
D_H = 128
D_O = 64
BLK = 2000

NCHUNK = 4
CHUNK_N = 12500
BAND = 12544
ACC_ROWS = 12560
DUMP = 12544
EPT = E // 16
SEG = 1000
NSEG = EPT // SEG
SEG_PAD = 1024
CLEN = 1024
ZROWS = BAND // 16
WB = 16


def _messages(src0, dst0, src1, dst1, w, code_table, provider_table):
    mesh = plsc.VectorSubcoreMesh(core_axis_name="c", subcore_axis_name="s")

    @functools.partial(
        pl.kernel,
        out_type=[jax.ShapeDtypeStruct((NCHUNK * BAND, 128), jnp.float32),
                  jax.ShapeDtypeStruct((NCHUNK * BAND, 128), jnp.float32)],
        mesh=mesh,
        compiler_params=pltpu.CompilerParams(needs_layout_passes=False,
                                             has_side_effects=True,
                                             use_tc_tiling_on_sc=False),
        scratch_types=[
            pltpu.VMEM((SEG_PAD,), jnp.int32),
            pltpu.VMEM((SEG_PAD,), jnp.int32),
            pltpu.VMEM((SEG_PAD,), jnp.float32),
            pltpu.VMEM((CLEN,), jnp.int32),
            pltpu.VMEM((CLEN,), jnp.int32),
            pltpu.VMEM((CLEN,), jnp.float32),
            pltpu.VMEM((16, 128), jnp.float32),
            pltpu.VMEM((16,), jnp.int32),
            pltpu.VMEM((16,), jnp.int32),
            pltpu.VMEM((16, 16), jnp.float32),
            pltpu.VMEM((1024,), jnp.int32),
            pltpu.VMEM((WB, 128), jnp.float32),
            pltpu.VMEM((WB, 16), jnp.float32),
            pltpu.VMEM_SHARED((ACC_ROWS, 128), jnp.float32),
            pltpu.VMEM_SHARED((ACC_ROWS, 16), jnp.float32),
            pltpu.SemaphoreType.DMA,
        ],
    )
    def k(src0_h, dst0_h, src1_h, dst1_h, w_h, ctab_h, ptab_h,
          out0_h, out1_h,
          src_s, dst_s, w_s, srcc, dstc, wc, gbuf, gidx, sidx, deg_stage,
          probe, nbuf, dbuf, acc_msg, acc_deg, sem):
        core = lax.axis_index("c")
        sub = lax.axis_index("s")
        iota = lax.iota(jnp.int32, 16)
        zeros16i = jnp.zeros((16,), jnp.int32)
        zeros16f = jnp.zeros((16,), jnp.float32)

        def init_srcc(i, c):
            srcc[pl.ds(i * 16, 16)] = zeros16i
            return c
        lax.fori_loop(0, CLEN // 16, init_srcc, 0)
        for t16 in (SEG, SEG_PAD - 16):
            dst_s[pl.ds(t16, 16)] = jnp.full((16,), -1, jnp.int32)
        for r in range(16):
            deg_stage[r, pl.ds(0, 16)] = zeros16f

        ebase = sub * EPT

        for d in range(2):
            sh = (src0_h, src1_h)[d]
            dh = (dst0_h, dst1_h)[d]
            th = (ctab_h, ptab_h)[d]
            oh = (out0_h, out1_h)[d]

            def task(local, carry):
                chunk = core * 2 + local
                chunk_lo = chunk * CHUNK_N
                band_lo = chunk * BAND
                for r2 in range(WB):
                    for q2 in range(8):
                        nbuf[r2, pl.ds(q2 * 16, 16)] = zeros16f
                    dbuf[r2, pl.ds(0, 16)] = zeros16f

                def zb(b, cz):
                    r0 = sub * ZROWS + b * WB
                    pltpu.sync_copy(nbuf, acc_msg.at[pl.ds(r0, WB)])
                    pltpu.sync_copy(dbuf, acc_deg.at[pl.ds(r0, WB)])
                    return cz

                lax.fori_loop(0, ZROWS // WB, zb, 0)
                plsc.subcore_barrier()

                def seg(s, c0):
                    sbase = ebase + s * SEG
                    pltpu.sync_copy(sh.at[pl.ds(sbase, SEG)],
                                    src_s.at[pl.ds(0, SEG)])
                    pltpu.sync_copy(dh.at[pl.ds(sbase, SEG)],
                                    dst_s.at[pl.ds(0, SEG)])
                    pltpu.sync_copy(w_h.at[pl.ds(sbase, SEG)],
                                    w_s.at[pl.ds(0, SEG)])

                    def compact(i, cnt):
                        off = i * 16
                        dv = dst_s[pl.ds(off, 16)] - chunk_lo
                        sv = src_s[pl.ds(off, 16)]
                        wv = w_s[pl.ds(off, 16)]
                        m = (dv >= 0) & (dv < CHUNK_N)
                        mi = jnp.where(m, 1, 0)
                        pos = cnt + plsc.cumsum(mi) - 1
                        plsc.store_scatter(dstc, [pos], dv, mask=m)
                        plsc.store_scatter(srcc, [pos], sv, mask=m)
                        plsc.store_scatter(wc, [pos], wv, mask=m)
                        return cnt + jnp.sum(mi)

                    cnt = lax.fori_loop(0, (SEG + 15) // 16, compact,
                                        jnp.int32(0))

                    def proc(j, c1):
                        jo = j * 16
                        valid = (jo + iota) < cnt
                        idxv = jnp.where(valid, dstc[pl.ds(jo, 16)], DUMP)
                        sidx[pl.ds(0, 16)] = idxv
                        gidx[pl.ds(0, 16)] = srcc[pl.ds(jo, 16)]
                        wvec = jnp.where(valid, wc[pl.ds(jo, 16)],
                                         jnp.float32(0.0))
                        pltpu.async_copy(th.at[gidx], gbuf, sem).wait()
                        for r in range(16):
                            w_r = wvec[r]
                            for q in range(8):
                                gbuf[r, pl.ds(q * 16, 16)] = (
                                    gbuf[r, pl.ds(q * 16, 16)] * w_r)
                        plsc.store_scatter(deg_stage, [iota, zeros16i], wvec)
                        hidx = idxv & 1023
                        plsc.store_scatter(probe, [hidx], iota)
                        rb = plsc.load_gather(probe, [hidx])
                        dup = jnp.sum(jnp.where(rb != iota, 1, 0)) > 0

                        @pl.when(jnp.logical_not(dup))
                        def _():
                            pltpu.sync_copy(gbuf, acc_msg.at[sidx],
                                            add=True)
                            pltpu.sync_copy(deg_stage, acc_deg.at[sidx],
                                            add=True)

                        @pl.when(dup)
                        def _():
                            def slow(r, cs):
                                sidx[pl.ds(0, 16)] = jnp.where(
                                    iota == r, idxv, DUMP)
                                plsc.store_scatter(
                                    deg_stage, [iota, zeros16i],
                                    jnp.where(iota == r, wvec,
                                              jnp.float32(0.0)))
                                pltpu.sync_copy(gbuf, acc_msg.at[sidx],
                                                add=True)
                                pltpu.sync_copy(deg_stage,
                                                acc_deg.at[sidx],
                                                add=True)
                                return cs
                            lax.fori_loop(0, 16, slow, 0)
                        return c1

                    nblk = (cnt + 15) // 16
                    lax.fori_loop(0, nblk, proc, 0)
                    return c0

                lax.fori_loop(0, NSEG, seg, 0)
                plsc.subcore_barrier()

                def wb(b, c2):
                    r0 = sub * ZROWS + b * WB
                    pltpu.sync_copy(acc_msg.at[pl.ds(r0, WB)], nbuf)
                    pltpu.sync_copy(acc_deg.at[pl.ds(r0, WB)], dbuf)

                    def row(r, c3):
                        dv = dbuf[r, pl.ds(0, 16)]
                        invv = 1.0 / (dv + 1e-8)
                        inv = invv[0]
                        for q in range(8):
                            nbuf[r, pl.ds(q * 16, 16)] = (
                                nbuf[r, pl.ds(q * 16, 16)] * inv)
                        return c3

                    lax.fori_loop(0, WB, row, 0)
                    pltpu.sync_copy(nbuf, oh.at[pl.ds(band_lo + r0, WB)])
                    return c2

                lax.fori_loop(0, ZROWS // WB, wb, 0)
                plsc.subcore_barrier()
                return carry

            lax.fori_loop(0, 2, task, 0)

    return k(src0, dst0, src1, dst1, w, code_table, provider_table)


def _mlp_body(x_ref, m_ref, w1a_ref, w1b_ref, b1_ref, g1_ref, be1_ref,
              w2_ref, b2_ref, g2_ref, be2_ref, o_ref):
    x = x_ref[...]
    m = m_ref[...]
    h = (jnp.dot(x, w1a_ref[...], preferred_element_type=jnp.float32)
         + jnp.dot(m, w1b_ref[...], preferred_element_type=jnp.float32)
         + b1_ref[...])
    mu = jnp.mean(h, axis=-1, keepdims=True)
    var = jnp.mean((h - mu) ** 2, axis=-1, keepdims=True)
    h = (h - mu) / jnp.sqrt(var + 1e-5) * g1_ref[...] + be1_ref[...]
    h = jnp.maximum(h, 0.0)
    o = jnp.dot(h, w2_ref[...], preferred_element_type=jnp.float32) + b2_ref[...]
    mu2 = jnp.mean(o, axis=-1, keepdims=True)
    var2 = jnp.mean((o - mu2) ** 2, axis=-1, keepdims=True)
    o_ref[...] = (o - mu2) / jnp.sqrt(var2 + 1e-5) * g2_ref[...] + be2_ref[...]


def _mlp(x, m, W1, b1, g1, be1, W2, b2, g2, be2):
    w1a = W1[:D_IN]
    w1b = W1[D_IN:]
    row = lambda i: (i, 0)
    fixed = lambda i: (0, 0)
    return pl.pallas_call(
        _mlp_body,
        grid=(N_NODES // BLK,),
        in_specs=[
            pl.BlockSpec((BLK, D_IN), row),
            pl.BlockSpec((BLK, D_IN), row),
            pl.BlockSpec((D_IN, D_H), fixed),
            pl.BlockSpec((D_IN, D_H), fixed),
            pl.BlockSpec((1, D_H), fixed),
            pl.BlockSpec((1, D_H), fixed),
            pl.BlockSpec((1, D_H), fixed),
            pl.BlockSpec((D_H, D_O), fixed),
            pl.BlockSpec((1, D_O), fixed),
            pl.BlockSpec((1, D_O), fixed),
            pl.BlockSpec((1, D_O), fixed),
        ],
        out_specs=pl.BlockSpec((BLK, D_O), row),
        out_shape=jax.ShapeDtypeStruct((N_NODES, D_O), jnp.float32),
    )(x, m, w1a, w1b, b1.reshape(1, -1), g1.reshape(1, -1), be1.reshape(1, -1),
      W2, b2.reshape(1, -1), g2.reshape(1, -1), be2.reshape(1, -1))


def kernel(provider_code_edges, code_provider_edges, edge_weights,
           provider_table, code_table,
           pW1, pb1, pg1, pbe1, pW2, pb2, pg2, pbe2,
           cW1, cb1, cg1, cbe1, cW2, cb2, cg2, cbe2):
    src0 = code_provider_edges[0].astype(jnp.int32)
    dst0 = code_provider_edges[1].astype(jnp.int32)
    src1 = provider_code_edges[0].astype(jnp.int32)
    dst1 = provider_code_edges[1].astype(jnp.int32)
    prov_band, code_band = _messages(src0, dst0, src1, dst1, edge_weights,
                                     code_table, provider_table)
    prov_msg = prov_band.reshape(NCHUNK, BAND, D_IN)[:, :CHUNK_N].reshape(
        N_NODES, D_IN)
    code_msg = code_band.reshape(NCHUNK, BAND, D_IN)[:, :CHUNK_N].reshape(
        N_NODES, D_IN)
    provider_out = _mlp(provider_table, prov_msg,
                        pW1, pb1, pg1, pbe1, pW2, pb2, pg2, pbe2)
    code_out = _mlp(code_table, code_msg,
                    cW1, cb1, cg1, cbe1, cW2, cb2, cg2, cbe2)
    return (provider_out, code_out)
